# baseline (speedup 1.0000x reference)
import jax
import jax.numpy as jnp
from jax.experimental import pallas as pl
from jax.experimental.pallas import tpu as pltpu

GRID_N = 27
PATCHES = GRID_N * GRID_N
BATCH_BLOCK = 1


def _body(x_ref, row_ref, col_ref, o_ref, pos_ref):
    i = pl.program_id(0)

    @pl.when(i == 0)
    def _():
        row = row_ref[...]
        col = col_ref[...]
        d = row.shape[-1]
        rr = jnp.reshape(
            jax.lax.broadcast_in_dim(row, (GRID_N, GRID_N, d), (0, 2)),
            (PATCHES, d),
        )
        cc = jnp.reshape(
            jax.lax.broadcast_in_dim(col, (GRID_N, GRID_N, d), (1, 2)),
            (PATCHES, d),
        )
        pos_ref[...] = rr + cc

    o_ref[...] = x_ref[...] + pos_ref[...][None, :, :]


def kernel(x, row_emb, col_emb):
    b, p, d = x.shape
    grid = (b // BATCH_BLOCK,)
    return pl.pallas_call(
        _body,
        grid=grid,
        in_specs=[
            pl.BlockSpec((BATCH_BLOCK, p, d), lambda i: (i, 0, 0)),
            pl.BlockSpec((GRID_N, d), lambda i: (0, 0)),
            pl.BlockSpec((GRID_N, d), lambda i: (0, 0)),
        ],
        out_specs=pl.BlockSpec((BATCH_BLOCK, p, d), lambda i: (i, 0, 0)),
        out_shape=jax.ShapeDtypeStruct(x.shape, x.dtype),
        scratch_shapes=[pltpu.VMEM((PATCHES, d), x.dtype)],
        compiler_params=pltpu.CompilerParams(
            dimension_semantics=("arbitrary",),
        ),
    )(x, row_emb, col_emb)


# speedup vs baseline: 1.0178x; 1.0178x over previous
import jax
import jax.numpy as jnp
from jax.experimental import pallas as pl
from jax.experimental.pallas import tpu as pltpu

GRID_N = 27
PATCHES = GRID_N * GRID_N
BATCH_BLOCK = 2
DEPTH = 4


def _body(x_hbm, row_ref, col_ref, o_hbm, pos_ref, in_bufs, out_bufs,
          in_sems, out_sems):
    d = row_ref.shape[-1]
    row = row_ref[...]
    col = col_ref[...]
    rr = jnp.reshape(
        jax.lax.broadcast_in_dim(row, (GRID_N, GRID_N, d), (0, 2)),
        (PATCHES, d),
    )
    cc = jnp.reshape(
        jax.lax.broadcast_in_dim(col, (GRID_N, GRID_N, d), (1, 2)),
        (PATCHES, d),
    )
    pos_ref[...] = rr + cc

    n_chunks = x_hbm.shape[0] // BATCH_BLOCK

    def start_in(c):
        k = c % DEPTH
        pltpu.make_async_copy(
            x_hbm.at[pl.ds(c * BATCH_BLOCK, BATCH_BLOCK)],
            in_bufs.at[k],
            in_sems.at[k],
        ).start()

    def wait_in(c):
        k = c % DEPTH
        pltpu.make_async_copy(
            x_hbm.at[pl.ds(c * BATCH_BLOCK, BATCH_BLOCK)],
            in_bufs.at[k],
            in_sems.at[k],
        ).wait()

    def start_out(c):
        k = c % DEPTH
        pltpu.make_async_copy(
            out_bufs.at[k],
            o_hbm.at[pl.ds(c * BATCH_BLOCK, BATCH_BLOCK)],
            out_sems.at[k],
        ).start()

    def wait_out(c):
        k = c % DEPTH
        pltpu.make_async_copy(
            out_bufs.at[k],
            o_hbm.at[pl.ds(c * BATCH_BLOCK, BATCH_BLOCK)],
            out_sems.at[k],
        ).wait()

    for k in range(DEPTH):
        start_in(k)

    pos = pos_ref[...][None, :, :]
    for c in range(n_chunks):
        k = c % DEPTH
        wait_in(c)
        if c >= DEPTH:
            wait_out(c - DEPTH)
        out_bufs[k] = in_bufs[k] + pos
        start_out(c)
        if c + DEPTH < n_chunks:
            start_in(c + DEPTH)

    for c in range(max(0, n_chunks - DEPTH), n_chunks):
        wait_out(c)


def kernel(x, row_emb, col_emb):
    b, p, d = x.shape
    return pl.pallas_call(
        _body,
        in_specs=[
            pl.BlockSpec(memory_space=pl.ANY),
            pl.BlockSpec(memory_space=pltpu.MemorySpace.VMEM),
            pl.BlockSpec(memory_space=pltpu.MemorySpace.VMEM),
        ],
        out_specs=pl.BlockSpec(memory_space=pl.ANY),
        out_shape=jax.ShapeDtypeStruct(x.shape, x.dtype),
        scratch_shapes=[
            pltpu.VMEM((p, d), x.dtype),
            pltpu.VMEM((DEPTH, BATCH_BLOCK, p, d), x.dtype),
            pltpu.VMEM((DEPTH, BATCH_BLOCK, p, d), x.dtype),
            pltpu.SemaphoreType.DMA((DEPTH,)),
            pltpu.SemaphoreType.DMA((DEPTH,)),
        ],
    )(x, row_emb, col_emb)
